# baseline (speedup 1.0000x reference)
import jax
import jax.numpy as jnp
from jax import lax
from jax.experimental import pallas as pl
from jax.experimental.pallas import tpu as pltpu
from jax.experimental.pallas import tpu_sc as plsc

NUM_CORES = 2
NUM_SUBCORES = 16
LANES = 16
NUM_WORKERS = NUM_CORES * NUM_SUBCORES

BATCH = 16384
FACTOR = 32
B_PER_W = BATCH // NUM_WORKERS
GROUPS = B_PER_W // LANES


def _bpr_body(user_hbm, item_i_hbm, item_j_hbm, utab_hbm, itab_hbm,
              out_i_hbm, out_j_hbm,
              uidx_v, iidx_v, jidx_v, urows_v, irows_v, jrows_v,
              acc_i_v, acc_j_v, tbuf_i, tbuf_j, sem0, sem1, sem2):
  wid = lax.axis_index("s") * NUM_CORES + lax.axis_index("c")
  base = wid * B_PER_W

  c0 = pltpu.async_copy(user_hbm.at[pl.ds(base, B_PER_W)], uidx_v, sem0)
  c1 = pltpu.async_copy(item_i_hbm.at[pl.ds(base, B_PER_W)], iidx_v, sem1)
  c2 = pltpu.async_copy(item_j_hbm.at[pl.ds(base, B_PER_W)], jidx_v, sem2)
  c0.wait()
  c1.wait()
  c2.wait()

  g0 = pltpu.async_copy(utab_hbm.at[uidx_v], urows_v, sem0)
  g1 = pltpu.async_copy(itab_hbm.at[iidx_v], irows_v, sem1)
  g2 = pltpu.async_copy(itab_hbm.at[jidx_v], jrows_v, sem2)
  g0.wait()
  g1.wait()
  g2.wait()

  lane_iota = lax.iota(jnp.int32, LANES)

  def group(g, carry):
    base_row = g * LANES
    for r in range(LANES):
      row = base_row + r
      u0 = urows_v[row, pl.ds(0, LANES)]
      u1 = urows_v[row, pl.ds(LANES, LANES)]
      i0 = irows_v[row, pl.ds(0, LANES)]
      i1 = irows_v[row, pl.ds(LANES, LANES)]
      j0 = jrows_v[row, pl.ds(0, LANES)]
      j1 = jrows_v[row, pl.ds(LANES, LANES)]
      p_i = u0 * i0 + u1 * i1
      p_j = u0 * j0 + u1 * j1
      tcol = lane_iota * LANES + r
      plsc.store_scatter(tbuf_i, [tcol], p_i)
      plsc.store_scatter(tbuf_j, [tcol], p_j)
    acc_i = tbuf_i[pl.ds(0, LANES)]
    acc_j = tbuf_j[pl.ds(0, LANES)]
    for k in range(1, LANES):
      acc_i = acc_i + tbuf_i[pl.ds(k * LANES, LANES)]
      acc_j = acc_j + tbuf_j[pl.ds(k * LANES, LANES)]
    acc_i_v[pl.ds(base_row, LANES)] = acc_i
    acc_j_v[pl.ds(base_row, LANES)] = acc_j
    return carry

  lax.fori_loop(0, GROUPS, group, 0)

  pltpu.sync_copy(acc_i_v, out_i_hbm.at[pl.ds(base, B_PER_W)])
  pltpu.sync_copy(acc_j_v, out_j_hbm.at[pl.ds(base, B_PER_W)])


@jax.jit
def _bpr(user, item_i, item_j, embed_user_weight, embed_item_weight):
  mesh = plsc.VectorSubcoreMesh(core_axis_name="c", subcore_axis_name="s")
  f = pl.kernel(
      _bpr_body,
      out_type=(
          jax.ShapeDtypeStruct((BATCH,), jnp.float32),
          jax.ShapeDtypeStruct((BATCH,), jnp.float32),
      ),
      mesh=mesh,
      scratch_types=[
          pltpu.VMEM((B_PER_W,), jnp.int32),
          pltpu.VMEM((B_PER_W,), jnp.int32),
          pltpu.VMEM((B_PER_W,), jnp.int32),
          pltpu.VMEM((B_PER_W, FACTOR), jnp.float32),
          pltpu.VMEM((B_PER_W, FACTOR), jnp.float32),
          pltpu.VMEM((B_PER_W, FACTOR), jnp.float32),
          pltpu.VMEM((B_PER_W,), jnp.float32),
          pltpu.VMEM((B_PER_W,), jnp.float32),
          pltpu.VMEM((LANES * LANES,), jnp.float32),
          pltpu.VMEM((LANES * LANES,), jnp.float32),
          pltpu.SemaphoreType.DMA,
          pltpu.SemaphoreType.DMA,
          pltpu.SemaphoreType.DMA,
      ],
      compiler_params=pltpu.CompilerParams(
          needs_layout_passes=False, use_tc_tiling_on_sc=False),
      name="bpr_sc",
  )
  return f(user, item_i, item_j, embed_user_weight, embed_item_weight)


def kernel(user, item_i, item_j, embed_user_weight, embed_item_weight):
  user = user.astype(jnp.int32)
  item_i = item_i.astype(jnp.int32)
  item_j = item_j.astype(jnp.int32)
  return _bpr(user, item_i, item_j, embed_user_weight, embed_item_weight)
